# baseline (speedup 1.0000x reference)
import functools

import jax
import jax.numpy as jnp
from jax import lax
from jax.experimental import pallas as pl
from jax.experimental.pallas import tpu as pltpu
from jax.experimental.pallas import tpu_sc as plsc

_NSEM = 8


def kernel(class_id, table):
    (B,) = class_id.shape
    V, D = table.shape
    info = plsc.get_sparse_core_info()
    NC, NS = info.num_cores, info.num_subcores
    NW = NC * NS
    b_per_w = B // NW

    q2 = class_id.astype(jnp.int32).reshape(NW, b_per_w)
    mesh = plsc.VectorSubcoreMesh(core_axis_name="c", subcore_axis_name="s")

    @functools.partial(
        pl.kernel,
        mesh=mesh,
        out_type=jax.ShapeDtypeStruct((B, D), jnp.float32),
        scratch_types=[
            pltpu.VMEM((b_per_w,), jnp.int32),
            pltpu.VMEM((b_per_w, D), jnp.float32),
            [pltpu.SemaphoreType.DMA] * _NSEM,
        ],
    )
    def emb(table_hbm, q_hbm, out_hbm, q_v, rows_v, sems):
        wid = lax.axis_index("s") * NC + lax.axis_index("c")
        pltpu.sync_copy(q_hbm.at[wid], q_v)

        def issue(g, carry):
            vec = q_v[pl.ds(g * 16, 16)]
            for k in range(16):
                pltpu.make_async_copy(
                    table_hbm.at[pl.ds(vec[k], 1)],
                    rows_v.at[pl.ds(g * 16 + k, 1)],
                    sems[k % _NSEM],
                ).start()
            return carry

        lax.fori_loop(0, b_per_w // 16, issue, 0)

        def drain(g, carry):
            for k in range(16):
                pltpu.make_async_copy(
                    table_hbm.at[pl.ds(0, 1)],
                    rows_v.at[pl.ds(g * 16 + k, 1)],
                    sems[k % _NSEM],
                ).wait()
            return carry

        lax.fori_loop(0, b_per_w // 16, drain, 0)
        pltpu.sync_copy(rows_v, out_hbm.at[pl.ds(wid * b_per_w, b_per_w)])

    return emb(table, q2)
